# baseline (speedup 1.0000x reference)
import functools

import jax
import jax.numpy as jnp
from jax.experimental import pallas as pl
from jax.experimental.pallas import tpu as pltpu

_CONF = 0.9
_NSPLIT = 2
_WB = 2560


def _body(nblk, steps, B, V, Wb, *refs):
    out_refs = refs[:_NSPLIT]
    t_ref = refs[_NSPLIT]
    oh_refs = refs[_NSPLIT + 1:2 * _NSPLIT + 1]
    res_ref = refs[2 * _NSPLIT + 1]
    accw_ref, acck_ref, g_ref, oht_ref = refs[2 * _NSPLIT + 2:]

    k = pl.program_id(0)

    @pl.when(k == 0)
    def _init():
        accw_ref[0, 0] = 0.0
        acck_ref[0, 0] = 0.0
        g_ref[...] = jnp.zeros_like(g_ref)
        oht_ref[...] = jnp.zeros_like(oht_ref)

    tcol = t_ref[...]
    accw = 0.0
    acck = 0.0
    for i in range(_NSPLIT):
        bi_log = i * steps + k
        bi = jnp.minimum(bi_log, nblk - 1)
        live = bi_log < nblk
        x = out_refs[i][...]
        oh = oh_refs[i][...]
        col = jax.lax.broadcasted_iota(jnp.int32, (1, Wb), 1) + bi * Wb
        valid = (col < V) & live

        colsum = jnp.sum(x, axis=0, keepdims=True)
        accw += jnp.sum(jnp.where(valid, colsum * oh, 0.0))

        safe = jnp.where(oh > 0, oh, 1.0)
        acck += jnp.sum(jnp.where(valid & (oh > 0), oh * jnp.log(safe), 0.0))

        cols2 = jax.lax.broadcasted_iota(jnp.int32, (B, Wb), 1) + bi * Wb
        mask = (cols2 == tcol) & live
        g_ref[...] += jnp.sum(jnp.where(mask, x, 0.0), axis=1, keepdims=True)
        ohb = jnp.broadcast_to(oh, (B, Wb))
        oht_ref[...] += jnp.sum(jnp.where(mask, ohb, 0.0), axis=1, keepdims=True)

    accw_ref[0, 0] += accw
    acck_ref[0, 0] += acck

    @pl.when(k == steps - 1)
    def _fin():
        g = g_ref[...]
        oht = oht_ref[...]
        safe_t = jnp.where(oht > 0, oht, 1.0)
        xlogy_t = jnp.where(oht > 0, oht * jnp.log(safe_t), 0.0)
        corr = _CONF * jnp.log(_CONF) - _CONF * g - xlogy_t + oht * g
        res_ref[0, 0] = (B * acck_ref[0, 0] - accw_ref[0, 0] + jnp.sum(corr))


def kernel(output, target, one_hot):
    B, V = output.shape
    Wb = _WB
    nblk = pl.cdiv(V, Wb)
    steps = pl.cdiv(nblk, _NSPLIT)

    t2 = target.reshape(B, 1)
    oh2 = one_hot.reshape(1, V)

    def mk_map(i):
        return lambda k: (0, jnp.minimum(i * steps + k, nblk - 1))

    in_specs = (
        [pl.BlockSpec((B, Wb), mk_map(i)) for i in range(_NSPLIT)]
        + [pl.BlockSpec((B, 1), lambda k: (0, 0))]
        + [pl.BlockSpec((1, Wb), mk_map(i)) for i in range(_NSPLIT)]
    )

    res = pl.pallas_call(
        functools.partial(_body, nblk, steps, B, V, Wb),
        grid=(steps,),
        in_specs=in_specs,
        out_specs=pl.BlockSpec(memory_space=pltpu.SMEM),
        out_shape=jax.ShapeDtypeStruct((1, 1), jnp.float32),
        scratch_shapes=[
            pltpu.SMEM((1, 1), jnp.float32),
            pltpu.SMEM((1, 1), jnp.float32),
            pltpu.VMEM((B, 1), jnp.float32),
            pltpu.VMEM((B, 1), jnp.float32),
        ],
        compiler_params=pltpu.CompilerParams(
            dimension_semantics=("arbitrary",),
        ),
    )(*([output] * _NSPLIT + [t2] + [oh2] * _NSPLIT))
    return res[0, 0]
